# baseline (speedup 1.0000x reference)
import functools
import jax
import jax.numpy as jnp
from jax import lax
from jax.experimental import pallas as pl
from jax.experimental.pallas import tpu as pltpu
from jax.experimental.pallas import tpu_sc as plsc

B = 2
T = 8
BT = B * T
N = 10000
E = 160000
D = 128
H = 128

NC = 2
NS = 16
LANES = 16

EPT = E // NS
K = 80
NCH = EPT // K
GN = BT // 2
SPT = GN // NC
RPT = N // NS

_mesh = lambda: plsc.VectorSubcoreMesh(
    core_axis_name="c", subcore_axis_name="s", num_cores=NC, num_subcores=NS)


def _sc_agg_body(base, xf_hbm, pidx_hbm, out_hbm,
                 pidx_t, rb0, rb1, agg_sh,
                 g0, g1, s0, s1, zsem, csem):
    cid = lax.axis_index("c")
    sid = lax.axis_index("s")

    pltpu.sync_copy(pidx_hbm.at[sid], pidx_t)

    def slice_body(j, carry):
        lbt = cid * SPT + j
        bt = base + lbt
        r0 = sid * RPT
        xfsl = xf_hbm.at[pl.ds(bt * N, N)]

        def gth(k, rbuf, sem):
            return pltpu.make_async_copy(
                xfsl.at[pidx_t.at[k].at[0]], rbuf, sem)

        def sct(k, rbuf, sem):
            return pltpu.make_async_copy(
                rbuf, agg_sh.at[pidx_t.at[k].at[1]], sem)

        pltpu.make_async_copy(xfsl.at[pl.ds(r0, RPT)],
                              agg_sh.at[pl.ds(r0, RPT)], zsem).start()
        pltpu.make_async_copy(xfsl.at[pl.ds(r0, RPT)],
                              agg_sh.at[pl.ds(r0, RPT)], zsem).wait()
        plsc.subcore_barrier()

        gth(0, rb0, g0).start()
        gth(1, rb1, g1).start()

        def pair(i, c2):
            k0 = 2 * i
            k1 = 2 * i + 1
            gth(k0, rb0, g0).wait()
            sct(k0, rb0, s0).start(add=True)
            sct(k0, rb0, s0).wait()
            gth(k0 + 2, rb0, g0).start()
            gth(k1, rb1, g1).wait()
            sct(k1, rb1, s1).start(add=True)
            sct(k1, rb1, s1).wait()

            @pl.when(k1 + 2 < NCH)
            def _():
                gth(k1 + 2, rb1, g1).start()
            return c2
        lax.fori_loop(0, (NCH - 1) // 2, pair, 0)
        gth(NCH - 1, rb0, g0).wait()
        sct(NCH - 1, rb0, s0).start(add=True)
        sct(NCH - 1, rb0, s0).wait()
        plsc.subcore_barrier()

        pltpu.make_async_copy(agg_sh.at[pl.ds(r0, RPT)],
                              out_hbm.at[pl.ds(lbt * N + r0, RPT)],
                              csem).start()
        pltpu.make_async_copy(agg_sh.at[pl.ds(r0, RPT)],
                              out_hbm.at[pl.ds(lbt * N + r0, RPT)],
                              csem).wait()
        plsc.subcore_barrier()
        return carry

    lax.fori_loop(0, SPT, slice_body, 0)


def _make_sc_agg(base, interpret=False):
    return pl.kernel(
        functools.partial(_sc_agg_body, base),
        out_type=jax.ShapeDtypeStruct((GN * N, D), jnp.float32),
        mesh=_mesh(),
        compiler_params=pltpu.CompilerParams(needs_layout_passes=False, use_tc_tiling_on_sc=False),
        scratch_types=[
            pltpu.VMEM((NCH, 2, K), jnp.int32),
            pltpu.VMEM((K, D), jnp.float32),
            pltpu.VMEM((K, D), jnp.float32),
            pltpu.VMEM_SHARED((N, D), jnp.float32),
            pltpu.SemaphoreType.DMA,
            pltpu.SemaphoreType.DMA,
            pltpu.SemaphoreType.DMA,
            pltpu.SemaphoreType.DMA,
            pltpu.SemaphoreType.DMA,
            pltpu.SemaphoreType.DMA,
        ],
        interpret=interpret,
    )


def _sc_deg_body(dst_hbm, z1d_hbm, cnt_hbm, dst_all, deg1d):
    cid = lax.axis_index("c")
    sid = lax.axis_index("s")

    @pl.when(cid == 0)
    def _():
        pltpu.sync_copy(dst_hbm.at[pl.ds(sid * EPT, EPT)], dst_all)
        pltpu.sync_copy(z1d_hbm, deg1d)

        ones = jnp.full((LANES,), 1.0, jnp.float32)

        def it(i, c2):
            d = dst_all[pl.ds(i * LANES, LANES)]
            plsc.addupdate_scatter(deg1d, [d], ones)
            return c2
        lax.fori_loop(0, EPT // LANES, it, 0)
        pltpu.sync_copy(deg1d, cnt_hbm.at[sid])


def _make_sc_deg(interpret=False):
    return pl.kernel(
        _sc_deg_body,
        out_type=jax.ShapeDtypeStruct((NS, N), jnp.float32),
        mesh=_mesh(),
        compiler_params=pltpu.CompilerParams(needs_layout_passes=False, use_tc_tiling_on_sc=False),
        scratch_types=[
            pltpu.VMEM((EPT,), jnp.int32),
            pltpu.VMEM((N,), jnp.float32),
        ],
        interpret=interpret,
    )


def _sc_root_body(nsa_hbm, src_hbm, dst_hbm, z1d_hbm, seg_hbm,
                  nsa_v, src_all, dst_all, seg1d):
    cid = lax.axis_index("c")
    sid = lax.axis_index("s")

    pltpu.sync_copy(nsa_hbm.at[cid], nsa_v)
    pltpu.sync_copy(src_hbm.at[pl.ds(sid * EPT, EPT)], src_all)
    pltpu.sync_copy(dst_hbm.at[pl.ds(sid * EPT, EPT)], dst_all)
    pltpu.sync_copy(z1d_hbm, seg1d)

    def it(i, c2):
        s = src_all[pl.ds(i * LANES, LANES)]
        vals = plsc.load_gather(nsa_v, [s])
        d = dst_all[pl.ds(i * LANES, LANES)]
        plsc.addupdate_scatter(seg1d, [d], vals)
        return c2
    lax.fori_loop(0, EPT // LANES, it, 0)
    pltpu.sync_copy(seg1d, seg_hbm.at[cid * NS + sid])


def _make_sc_root(interpret=False):
    return pl.kernel(
        _sc_root_body,
        out_type=jax.ShapeDtypeStruct((B * NS, N), jnp.float32),
        mesh=_mesh(),
        compiler_params=pltpu.CompilerParams(needs_layout_passes=False, use_tc_tiling_on_sc=False),
        scratch_types=[
            pltpu.VMEM((N,), jnp.float32),
            pltpu.VMEM((EPT,), jnp.int32),
            pltpu.VMEM((EPT,), jnp.int32),
            pltpu.VMEM((N,), jnp.float32),
        ],
        interpret=interpret,
    )


_RB = 1000


def _deg_body(cnt_ref, c_ref, inv_ref):
    c = jnp.sum(cnt_ref[...], axis=0)
    c_ref[...] = c
    inv_ref[...] = 1.0 / (c + 1.0)


def _tc_deg(cnt_p, interpret=False):
    return pl.pallas_call(
        _deg_body,
        grid=(N // _RB,),
        in_specs=[pl.BlockSpec((NS, _RB, 1), lambda nb: (0, nb, 0))],
        out_specs=[
            pl.BlockSpec((_RB, 1), lambda nb: (nb, 0)),
            pl.BlockSpec((_RB, 1), lambda nb: (nb, 0)),
        ],
        out_shape=[
            jax.ShapeDtypeStruct((N, 1), jnp.float32),
            jax.ShapeDtypeStruct((N, 1), jnp.float32),
        ],
        interpret=interpret,
    )(cnt_p)


def _dense_body_dual(a_ref, inv_ref, wT_ref, b_ref, g_ref, be_ref,
                     o_ref, ob_ref):
    v = a_ref[0] * inv_ref[...]
    o = jnp.dot(v.astype(jnp.bfloat16), wT_ref[...],
                preferred_element_type=jnp.float32) + b_ref[...]
    mu = jnp.mean(o, axis=-1, keepdims=True)
    dlt = o - mu
    var = jnp.mean(dlt * dlt, axis=-1, keepdims=True)
    y = dlt * lax.rsqrt(var + 1e-5) * g_ref[...] + be_ref[...]
    r = jnp.maximum(y, 0.0)
    o_ref[0] = r
    if ob_ref is not None:
        ob_ref[0] = r.astype(jnp.bfloat16)


def _dense_body(a_ref, inv_ref, wT_ref, b_ref, g_ref, be_ref, o_ref):
    _dense_body_dual(a_ref, inv_ref, wT_ref, b_ref, g_ref, be_ref,
                     o_ref, None)


def _tc_dense(agg3, inv, W, b, g, be, dual=False, interpret=False):
    wspec = pl.BlockSpec((D, H), lambda bt, nb: (0, 0))
    vspec = pl.BlockSpec((1, H), lambda bt, nb: (0, 0))
    ospec = pl.BlockSpec((1, _RB, H), lambda bt, nb: (bt, nb, 0))
    nbt = agg3.shape[0]
    return pl.pallas_call(
        _dense_body_dual if dual else _dense_body,
        grid=(nbt, N // _RB),
        in_specs=[
            pl.BlockSpec((1, _RB, D), lambda bt, nb: (bt, nb, 0)),
            pl.BlockSpec((_RB, 1), lambda bt, nb: (nb, 0)),
            wspec, vspec, vspec, vspec,
        ],
        out_specs=[ospec, ospec] if dual else ospec,
        out_shape=(
            [jax.ShapeDtypeStruct((nbt, N, H), jnp.float32),
             jax.ShapeDtypeStruct((nbt, N, H), jnp.bfloat16)]
            if dual else jax.ShapeDtypeStruct((nbt, N, H), jnp.float32)),
        interpret=interpret,
    )(agg3, inv, W.T.astype(jnp.bfloat16), b[None, :], g[None, :], be[None, :])


def _gru_step(x, h, WiT, WhT, bi, bh):
    gi = jnp.dot(x.astype(jnp.bfloat16), WiT,
                 preferred_element_type=jnp.float32) + bi
    gh = jnp.dot(h.astype(jnp.bfloat16), WhT,
                 preferred_element_type=jnp.float32) + bh
    r = jax.nn.sigmoid(gi[:, :H] + gh[:, :H])
    z = jax.nn.sigmoid(gi[:, H:2 * H] + gh[:, H:2 * H])
    n = jnp.tanh(gi[:, 2 * H:] + r * gh[:, 2 * H:])
    return (1.0 - z) * n + z * h


def _gru_body(x_ref, wi0_ref, wh0_ref, bi0_ref, bh0_ref,
              wi1_ref, wh1_ref, bi1_ref, bh1_ref, o_ref):
    Wi0 = wi0_ref[...]
    Wh0 = wh0_ref[...]
    Wi1 = wi1_ref[...]
    Wh1 = wh1_ref[...]
    bi0 = bi0_ref[...]
    bh0 = bh0_ref[...]
    bi1 = bi1_ref[...]
    bh1 = bh1_ref[...]
    h0 = jnp.zeros((_RB, H), jnp.float32)
    h1 = jnp.zeros((_RB, H), jnp.float32)
    for t in range(T):
        xt = x_ref[0, t]
        h0 = _gru_step(xt, h0, Wi0, Wh0, bi0, bh0)
        h1 = _gru_step(h0, h1, Wi1, Wh1, bi1, bh1)
    o_ref[0] = h1


def _tc_gru(x4, Wih0, Whh0, bih0, bhh0, Wih1, Whh1, bih1, bhh1,
            interpret=False):
    wspec = pl.BlockSpec((H, 3 * H), lambda b, nb: (0, 0))
    bspec = pl.BlockSpec((1, 3 * H), lambda b, nb: (0, 0))
    return pl.pallas_call(
        _gru_body,
        grid=(x4.shape[0], N // _RB),
        in_specs=[
            pl.BlockSpec((1, T, _RB, H), lambda b, nb: (b, 0, nb, 0)),
            wspec, wspec, bspec, bspec, wspec, wspec, bspec, bspec,
        ],
        out_specs=pl.BlockSpec((1, _RB, H), lambda b, nb: (b, nb, 0)),
        out_shape=jax.ShapeDtypeStruct((x4.shape[0], N, H), jnp.float32),
        interpret=interpret,
    )(x4, Wih0.T.astype(jnp.bfloat16), Whh0.T.astype(jnp.bfloat16),
      bih0[None, :], bhh0[None, :],
      Wih1.T.astype(jnp.bfloat16), Whh1.T.astype(jnp.bfloat16),
      bih1[None, :], bhh1[None, :])


def _heads_body(h0_ref, h1_ref, cnt_ref, wc1_ref, bc1_ref, wc2_ref, bc2_ref,
                wr1_ref, br1_ref, wr2_ref, br2_ref, we_ref, bee_ref,
                wn_ref, bn_ref, rc_ref, rr_ref, nsa_ref, base_ref):
    c = cnt_ref[...]
    weT = we_ref[...]
    for b, href in enumerate((h0_ref, h1_ref)):
        hb = href[0]
        z1 = jnp.maximum(
            jnp.dot(hb, wc1_ref[...], preferred_element_type=jnp.float32)
            + bc1_ref[...], 0.0)
        rc_ref[b] = jnp.dot(z1, wc2_ref[...],
                            preferred_element_type=jnp.float32) + bc2_ref[...]
        z2 = jnp.maximum(
            jnp.dot(hb, wr1_ref[...], preferred_element_type=jnp.float32)
            + br1_ref[...], 0.0)
        rr_ref[b] = jax.nn.sigmoid(
            jnp.dot(z2, wr2_ref[...], preferred_element_type=jnp.float32)
            + br2_ref[...])
        nsa_ref[b] = jnp.dot(hb, weT[:H], preferred_element_type=jnp.float32)
        nsb = jnp.dot(hb, weT[H:], preferred_element_type=jnp.float32)
        nss = jnp.dot(hb, wn_ref[...],
                      preferred_element_type=jnp.float32) + bn_ref[...]
        base_ref[b] = c * (nsb + bee_ref[...]) + nss


def _tc_heads(h, cnt, Wc1, bc1, Wc2, bc2, Wr1, br1, Wr2, br2,
              We, bee, Wn, bn, interpret=False):
    def full(shape):
        return pl.BlockSpec(shape, lambda nb: tuple(0 for _ in shape))
    return pl.pallas_call(
        _heads_body,
        grid=(N // _RB,),
        in_specs=[
            pl.BlockSpec((B, _RB, H), lambda nb: (0, nb, 0)),
            pl.BlockSpec((_RB, 1), lambda nb: (nb, 0)),
            full((H, H // 2)), full((1, H // 2)),
            full((H // 2, 4)), full((1, 4)),
            full((H, H // 2)), full((1, H // 2)),
            full((H // 2, 1)), full((1, 1)),
            full((2 * H, 1)), full((1, 1)),
            full((H, 1)), full((1, 1)),
        ],
        out_specs=[
            pl.BlockSpec((B, _RB, 4), lambda nb: (0, nb, 0)),
            pl.BlockSpec((B, _RB, 1), lambda nb: (0, nb, 0)),
            pl.BlockSpec((B, _RB, 1), lambda nb: (0, nb, 0)),
            pl.BlockSpec((B, _RB, 1), lambda nb: (0, nb, 0)),
        ],
        out_shape=[
            jax.ShapeDtypeStruct((B, N, 4), jnp.float32),
            jax.ShapeDtypeStruct((B, N, 1), jnp.float32),
            jax.ShapeDtypeStruct((B, N, 1), jnp.float32),
            jax.ShapeDtypeStruct((B, N, 1), jnp.float32),
        ],
        interpret=interpret,
    )(h0, h1, cnt, Wc1.T, bc1[None, :], Wc2.T, bc2[None, :],
      Wr1.T, br1[None, :], Wr2.T, br2[None, :],
      We.T, bee[None, :], Wn.T, bn[None, :])


def _softmax_body(seg_ref, base_ref, o_ref):
    l = jnp.sum(seg_ref[...], axis=1) + base_ref[...]
    m = jnp.max(l, axis=-1, keepdims=True)
    e = jnp.exp(l - m)
    o_ref[...] = e / jnp.sum(e, axis=-1, keepdims=True)


def _tc_softmax(seg, base, interpret=False):
    return pl.pallas_call(
        _softmax_body,
        out_shape=jax.ShapeDtypeStruct((B, N), jnp.float32),
        interpret=interpret,
    )(seg, base)


def kernel(x, edge_index, W_gcn1, b_gcn1, g1, be1, W_gcn2, b_gcn2, g2, be2,
           Wih0, Whh0, bih0, bhh0, Wih1, Whh1, bih1, bhh1, Wc1, bc1, Wc2, bc2,
           Wr1, br1, Wr2, br2, We, bee, Wn, bn):
    x = x.astype(jnp.float32)
    src = edge_index[0].astype(jnp.int32)
    dst = edge_index[1].astype(jnp.int32)
    pidx = jnp.stack([src.reshape(NS, NCH, K), dst.reshape(NS, NCH, K)],
                     axis=2)
    zeros_1d = jnp.zeros((N,), jnp.float32)

    sc_agg_a = _make_sc_agg(0)
    sc_agg_b = _make_sc_agg(GN)
    sc_deg = _make_sc_deg()
    sc_root = _make_sc_root()

    cnt_p = sc_deg(dst, zeros_1d).reshape(NS, N, 1)
    cnt, inv = _tc_deg(cnt_p)

    xf = x.reshape(BT * N, D)
    agg1a = sc_agg_a(xf, pidx).reshape(GN, N, D)
    agg1b = sc_agg_b(xf, pidx).reshape(GN, N, D)
    h1a = _tc_dense(agg1a, inv, W_gcn1, b_gcn1, g1, be1)
    h1b = _tc_dense(agg1b, inv, W_gcn1, b_gcn1, g1, be1)
    agg2a = sc_agg_a(h1a.reshape(GN * N, H), pidx).reshape(GN, N, D)
    h2a = _tc_dense(agg2a, inv, W_gcn2, b_gcn2, g2, be2)
    hga = _tc_gru(h2a.reshape(1, T, N, H), Wih0, Whh0, bih0, bhh0,
                  Wih1, Whh1, bih1, bhh1)
    agg2b = sc_agg_a(h1b.reshape(GN * N, H), pidx).reshape(GN, N, D)
    inv_b = inv + 0.0 * hga[0, :, :1]
    h2b = _tc_dense(agg2b, inv_b, W_gcn2, b_gcn2, g2, be2)
    hgb = _tc_gru(h2b.reshape(1, T, N, H), Wih0, Whh0, bih0, bhh0,
                  Wih1, Whh1, bih1, bhh1)

    rc, rr, nsa, base = _tc_heads(hga, hgb, cnt, Wc1, bc1, Wc2, bc2,
                                  Wr1, br1, Wr2, br2, We, bee, Wn, bn)
    hg = jnp.concatenate([hga, hgb], axis=0)

    seg = sc_root(nsa.reshape(B, N), src, dst, zeros_1d)
    root = _tc_softmax(seg.reshape(B, NS, N), base.reshape(B, N))

    return rc, rr.reshape(B, N), root, hg


# speedup vs baseline: 1.0211x; 1.0211x over previous
import functools
import jax
import jax.numpy as jnp
from jax import lax
from jax.experimental import pallas as pl
from jax.experimental.pallas import tpu as pltpu
from jax.experimental.pallas import tpu_sc as plsc

B = 2
T = 8
BT = B * T
N = 10000
E = 160000
D = 128
H = 128

NC = 2
NS = 16
LANES = 16

EPT = E // NS
K = 80
NCH = EPT // K
GN = BT // 2
SPT = GN // NC
RPT = N // NS

_mesh = lambda: plsc.VectorSubcoreMesh(
    core_axis_name="c", subcore_axis_name="s", num_cores=NC, num_subcores=NS)


def _sc_agg_body(base, xf_hbm, pidx_hbm, out_hbm,
                 pidx_t, rb0, rb1, agg_sh,
                 g0, g1, s0, s1, zsem, csem):
    cid = lax.axis_index("c")
    sid = lax.axis_index("s")

    pltpu.sync_copy(pidx_hbm.at[sid], pidx_t)

    def slice_body(j, carry):
        lbt = cid * SPT + j
        bt = base + lbt
        r0 = sid * RPT
        xfsl = xf_hbm.at[pl.ds(bt * N, N)]

        def gth(k, rbuf, sem):
            return pltpu.make_async_copy(
                xfsl.at[pidx_t.at[k].at[0]], rbuf, sem)

        def sct(k, rbuf, sem):
            return pltpu.make_async_copy(
                rbuf, agg_sh.at[pidx_t.at[k].at[1]], sem)

        pltpu.make_async_copy(xfsl.at[pl.ds(r0, RPT)],
                              agg_sh.at[pl.ds(r0, RPT)], zsem).start()
        pltpu.make_async_copy(xfsl.at[pl.ds(r0, RPT)],
                              agg_sh.at[pl.ds(r0, RPT)], zsem).wait()
        plsc.subcore_barrier()

        gth(0, rb0, g0).start()
        gth(1, rb1, g1).start()

        def pair(i, c2):
            k0 = 2 * i
            k1 = 2 * i + 1
            gth(k0, rb0, g0).wait()
            sct(k0, rb0, s0).start(add=True)
            sct(k0, rb0, s0).wait()
            gth(k0 + 2, rb0, g0).start()
            gth(k1, rb1, g1).wait()
            sct(k1, rb1, s1).start(add=True)
            sct(k1, rb1, s1).wait()

            @pl.when(k1 + 2 < NCH)
            def _():
                gth(k1 + 2, rb1, g1).start()
            return c2
        lax.fori_loop(0, (NCH - 1) // 2, pair, 0)
        gth(NCH - 1, rb0, g0).wait()
        sct(NCH - 1, rb0, s0).start(add=True)
        sct(NCH - 1, rb0, s0).wait()
        plsc.subcore_barrier()

        pltpu.make_async_copy(agg_sh.at[pl.ds(r0, RPT)],
                              out_hbm.at[pl.ds(lbt * N + r0, RPT)],
                              csem).start()
        pltpu.make_async_copy(agg_sh.at[pl.ds(r0, RPT)],
                              out_hbm.at[pl.ds(lbt * N + r0, RPT)],
                              csem).wait()
        plsc.subcore_barrier()
        return carry

    lax.fori_loop(0, SPT, slice_body, 0)


def _make_sc_agg(base, interpret=False):
    return pl.kernel(
        functools.partial(_sc_agg_body, base),
        out_type=jax.ShapeDtypeStruct((GN * N, D), jnp.float32),
        mesh=_mesh(),
        compiler_params=pltpu.CompilerParams(needs_layout_passes=False, use_tc_tiling_on_sc=False),
        scratch_types=[
            pltpu.VMEM((NCH, 2, K), jnp.int32),
            pltpu.VMEM((K, D), jnp.float32),
            pltpu.VMEM((K, D), jnp.float32),
            pltpu.VMEM_SHARED((N, D), jnp.float32),
            pltpu.SemaphoreType.DMA,
            pltpu.SemaphoreType.DMA,
            pltpu.SemaphoreType.DMA,
            pltpu.SemaphoreType.DMA,
            pltpu.SemaphoreType.DMA,
            pltpu.SemaphoreType.DMA,
        ],
        interpret=interpret,
    )


def _sc_deg_body(dst_hbm, z1d_hbm, cnt_hbm, dst_all, deg1d):
    cid = lax.axis_index("c")
    sid = lax.axis_index("s")

    @pl.when(cid == 0)
    def _():
        pltpu.sync_copy(dst_hbm.at[pl.ds(sid * EPT, EPT)], dst_all)
        pltpu.sync_copy(z1d_hbm, deg1d)

        ones = jnp.full((LANES,), 1.0, jnp.float32)

        def it(i, c2):
            d = dst_all[pl.ds(i * LANES, LANES)]
            plsc.addupdate_scatter(deg1d, [d], ones)
            return c2
        lax.fori_loop(0, EPT // LANES, it, 0)
        pltpu.sync_copy(deg1d, cnt_hbm.at[sid])


def _make_sc_deg(interpret=False):
    return pl.kernel(
        _sc_deg_body,
        out_type=jax.ShapeDtypeStruct((NS, N), jnp.float32),
        mesh=_mesh(),
        compiler_params=pltpu.CompilerParams(needs_layout_passes=False, use_tc_tiling_on_sc=False),
        scratch_types=[
            pltpu.VMEM((EPT,), jnp.int32),
            pltpu.VMEM((N,), jnp.float32),
        ],
        interpret=interpret,
    )


def _sc_root_body(nsa_hbm, src_hbm, dst_hbm, z1d_hbm, seg_hbm,
                  nsa_v, src_all, dst_all, seg1d):
    cid = lax.axis_index("c")
    sid = lax.axis_index("s")

    pltpu.sync_copy(nsa_hbm.at[cid], nsa_v)
    pltpu.sync_copy(src_hbm.at[pl.ds(sid * EPT, EPT)], src_all)
    pltpu.sync_copy(dst_hbm.at[pl.ds(sid * EPT, EPT)], dst_all)
    pltpu.sync_copy(z1d_hbm, seg1d)

    def it(i, c2):
        s = src_all[pl.ds(i * LANES, LANES)]
        vals = plsc.load_gather(nsa_v, [s])
        d = dst_all[pl.ds(i * LANES, LANES)]
        plsc.addupdate_scatter(seg1d, [d], vals)
        return c2
    lax.fori_loop(0, EPT // LANES, it, 0)
    pltpu.sync_copy(seg1d, seg_hbm.at[cid * NS + sid])


def _make_sc_root(interpret=False):
    return pl.kernel(
        _sc_root_body,
        out_type=jax.ShapeDtypeStruct((B * NS, N), jnp.float32),
        mesh=_mesh(),
        compiler_params=pltpu.CompilerParams(needs_layout_passes=False, use_tc_tiling_on_sc=False),
        scratch_types=[
            pltpu.VMEM((N,), jnp.float32),
            pltpu.VMEM((EPT,), jnp.int32),
            pltpu.VMEM((EPT,), jnp.int32),
            pltpu.VMEM((N,), jnp.float32),
        ],
        interpret=interpret,
    )


_RB = 1000


def _deg_body(cnt_ref, c_ref, inv_ref):
    c = jnp.sum(cnt_ref[...], axis=0)
    c_ref[...] = c
    inv_ref[...] = 1.0 / (c + 1.0)


def _tc_deg(cnt_p, interpret=False):
    return pl.pallas_call(
        _deg_body,
        grid=(N // _RB,),
        in_specs=[pl.BlockSpec((NS, _RB, 1), lambda nb: (0, nb, 0))],
        out_specs=[
            pl.BlockSpec((_RB, 1), lambda nb: (nb, 0)),
            pl.BlockSpec((_RB, 1), lambda nb: (nb, 0)),
        ],
        out_shape=[
            jax.ShapeDtypeStruct((N, 1), jnp.float32),
            jax.ShapeDtypeStruct((N, 1), jnp.float32),
        ],
        interpret=interpret,
    )(cnt_p)


def _dense_body_dual(a_ref, inv_ref, wT_ref, b_ref, g_ref, be_ref,
                     o_ref, ob_ref):
    v = a_ref[0] * inv_ref[...]
    o = jnp.dot(v.astype(jnp.bfloat16), wT_ref[...],
                preferred_element_type=jnp.float32) + b_ref[...]
    mu = jnp.mean(o, axis=-1, keepdims=True)
    dlt = o - mu
    var = jnp.mean(dlt * dlt, axis=-1, keepdims=True)
    y = dlt * lax.rsqrt(var + 1e-5) * g_ref[...] + be_ref[...]
    r = jnp.maximum(y, 0.0)
    o_ref[0] = r
    if ob_ref is not None:
        ob_ref[0] = r.astype(jnp.bfloat16)


def _dense_body(a_ref, inv_ref, wT_ref, b_ref, g_ref, be_ref, o_ref):
    _dense_body_dual(a_ref, inv_ref, wT_ref, b_ref, g_ref, be_ref,
                     o_ref, None)


def _tc_dense(agg3, inv, W, b, g, be, dual=False, interpret=False):
    wspec = pl.BlockSpec((D, H), lambda bt, nb: (0, 0))
    vspec = pl.BlockSpec((1, H), lambda bt, nb: (0, 0))
    ospec = pl.BlockSpec((1, _RB, H), lambda bt, nb: (bt, nb, 0))
    nbt = agg3.shape[0]
    return pl.pallas_call(
        _dense_body_dual if dual else _dense_body,
        grid=(nbt, N // _RB),
        in_specs=[
            pl.BlockSpec((1, _RB, D), lambda bt, nb: (bt, nb, 0)),
            pl.BlockSpec((_RB, 1), lambda bt, nb: (nb, 0)),
            wspec, vspec, vspec, vspec,
        ],
        out_specs=[ospec, ospec] if dual else ospec,
        out_shape=(
            [jax.ShapeDtypeStruct((nbt, N, H), jnp.float32),
             jax.ShapeDtypeStruct((nbt, N, H), jnp.bfloat16)]
            if dual else jax.ShapeDtypeStruct((nbt, N, H), jnp.float32)),
        interpret=interpret,
    )(agg3, inv, W.T.astype(jnp.bfloat16), b[None, :], g[None, :], be[None, :])


def _gru_step(x, h, WiT, WhT, bi, bh):
    gi = jnp.dot(x.astype(jnp.bfloat16), WiT,
                 preferred_element_type=jnp.float32) + bi
    gh = jnp.dot(h.astype(jnp.bfloat16), WhT,
                 preferred_element_type=jnp.float32) + bh
    r = jax.nn.sigmoid(gi[:, :H] + gh[:, :H])
    z = jax.nn.sigmoid(gi[:, H:2 * H] + gh[:, H:2 * H])
    n = jnp.tanh(gi[:, 2 * H:] + r * gh[:, 2 * H:])
    return (1.0 - z) * n + z * h


def _gru_body(a_ref, inv_ref, wT_ref, b_ref, g_ref, be_ref,
              wi0_ref, wh0_ref, bi0_ref, bh0_ref,
              wi1_ref, wh1_ref, bi1_ref, bh1_ref, o_ref):
    inv = inv_ref[...]
    wT = wT_ref[...]
    bb = b_ref[...]
    gg = g_ref[...]
    be = be_ref[...]
    Wi0 = wi0_ref[...]
    Wh0 = wh0_ref[...]
    Wi1 = wi1_ref[...]
    Wh1 = wh1_ref[...]
    bi0 = bi0_ref[...]
    bh0 = bh0_ref[...]
    bi1 = bi1_ref[...]
    bh1 = bh1_ref[...]
    h0 = jnp.zeros((_RB, H), jnp.float32)
    h1 = jnp.zeros((_RB, H), jnp.float32)
    for t in range(T):
        v = a_ref[0, t] * inv
        o = jnp.dot(v.astype(jnp.bfloat16), wT,
                    preferred_element_type=jnp.float32) + bb
        mu = jnp.mean(o, axis=-1, keepdims=True)
        dlt = o - mu
        var = jnp.mean(dlt * dlt, axis=-1, keepdims=True)
        xt = jnp.maximum(dlt * lax.rsqrt(var + 1e-5) * gg + be, 0.0)
        h0 = _gru_step(xt, h0, Wi0, Wh0, bi0, bh0)
        h1 = _gru_step(h0, h1, Wi1, Wh1, bi1, bh1)
    o_ref[0] = h1


def _tc_gru(a4, inv, W, b, g, be, Wih0, Whh0, bih0, bhh0,
            Wih1, Whh1, bih1, bhh1, interpret=False):
    wspec = pl.BlockSpec((H, 3 * H), lambda bq, nb: (0, 0))
    bspec = pl.BlockSpec((1, 3 * H), lambda bq, nb: (0, 0))
    return pl.pallas_call(
        _gru_body,
        grid=(1, N // _RB),
        in_specs=[
            pl.BlockSpec((1, T, _RB, D), lambda bq, nb: (bq, 0, nb, 0)),
            pl.BlockSpec((_RB, 1), lambda bq, nb: (nb, 0)),
            pl.BlockSpec((D, H), lambda bq, nb: (0, 0)),
            pl.BlockSpec((1, H), lambda bq, nb: (0, 0)),
            pl.BlockSpec((1, H), lambda bq, nb: (0, 0)),
            pl.BlockSpec((1, H), lambda bq, nb: (0, 0)),
            wspec, wspec, bspec, bspec, wspec, wspec, bspec, bspec,
        ],
        out_specs=pl.BlockSpec((1, _RB, H), lambda bq, nb: (bq, nb, 0)),
        out_shape=jax.ShapeDtypeStruct((1, N, H), jnp.float32),
        interpret=interpret,
    )(a4, inv, W.T.astype(jnp.bfloat16), b[None, :], g[None, :], be[None, :],
      Wih0.T.astype(jnp.bfloat16), Whh0.T.astype(jnp.bfloat16),
      bih0[None, :], bhh0[None, :],
      Wih1.T.astype(jnp.bfloat16), Whh1.T.astype(jnp.bfloat16),
      bih1[None, :], bhh1[None, :])


def _heads_body(h0_ref, h1_ref, cnt_ref, wc1_ref, bc1_ref, wc2_ref, bc2_ref,
                wr1_ref, br1_ref, wr2_ref, br2_ref, we_ref, bee_ref,
                wn_ref, bn_ref, rc_ref, rr_ref, nsa_ref, base_ref):
    c = cnt_ref[...]
    weT = we_ref[...]
    for b, href in enumerate((h0_ref, h1_ref)):
        hb = href[0]
        z1 = jnp.maximum(
            jnp.dot(hb, wc1_ref[...], preferred_element_type=jnp.float32)
            + bc1_ref[...], 0.0)
        rc_ref[b] = jnp.dot(z1, wc2_ref[...],
                            preferred_element_type=jnp.float32) + bc2_ref[...]
        z2 = jnp.maximum(
            jnp.dot(hb, wr1_ref[...], preferred_element_type=jnp.float32)
            + br1_ref[...], 0.0)
        rr_ref[b] = jax.nn.sigmoid(
            jnp.dot(z2, wr2_ref[...], preferred_element_type=jnp.float32)
            + br2_ref[...])
        nsa_ref[b] = jnp.dot(hb, weT[:H], preferred_element_type=jnp.float32)
        nsb = jnp.dot(hb, weT[H:], preferred_element_type=jnp.float32)
        nss = jnp.dot(hb, wn_ref[...],
                      preferred_element_type=jnp.float32) + bn_ref[...]
        base_ref[b] = c * (nsb + bee_ref[...]) + nss


def _tc_heads(h, cnt, Wc1, bc1, Wc2, bc2, Wr1, br1, Wr2, br2,
              We, bee, Wn, bn, interpret=False):
    def full(shape):
        return pl.BlockSpec(shape, lambda nb: tuple(0 for _ in shape))
    return pl.pallas_call(
        _heads_body,
        grid=(N // _RB,),
        in_specs=[
            pl.BlockSpec((B, _RB, H), lambda nb: (0, nb, 0)),
            pl.BlockSpec((_RB, 1), lambda nb: (nb, 0)),
            full((H, H // 2)), full((1, H // 2)),
            full((H // 2, 4)), full((1, 4)),
            full((H, H // 2)), full((1, H // 2)),
            full((H // 2, 1)), full((1, 1)),
            full((2 * H, 1)), full((1, 1)),
            full((H, 1)), full((1, 1)),
        ],
        out_specs=[
            pl.BlockSpec((B, _RB, 4), lambda nb: (0, nb, 0)),
            pl.BlockSpec((B, _RB, 1), lambda nb: (0, nb, 0)),
            pl.BlockSpec((B, _RB, 1), lambda nb: (0, nb, 0)),
            pl.BlockSpec((B, _RB, 1), lambda nb: (0, nb, 0)),
        ],
        out_shape=[
            jax.ShapeDtypeStruct((B, N, 4), jnp.float32),
            jax.ShapeDtypeStruct((B, N, 1), jnp.float32),
            jax.ShapeDtypeStruct((B, N, 1), jnp.float32),
            jax.ShapeDtypeStruct((B, N, 1), jnp.float32),
        ],
        interpret=interpret,
    )(h0, h1, cnt, Wc1.T, bc1[None, :], Wc2.T, bc2[None, :],
      Wr1.T, br1[None, :], Wr2.T, br2[None, :],
      We.T, bee[None, :], Wn.T, bn[None, :])


def _softmax_body(seg_ref, base_ref, o_ref):
    l = jnp.sum(seg_ref[...], axis=1) + base_ref[...]
    m = jnp.max(l, axis=-1, keepdims=True)
    e = jnp.exp(l - m)
    o_ref[...] = e / jnp.sum(e, axis=-1, keepdims=True)


def _tc_softmax(seg, base, interpret=False):
    return pl.pallas_call(
        _softmax_body,
        out_shape=jax.ShapeDtypeStruct((B, N), jnp.float32),
        interpret=interpret,
    )(seg, base)


def kernel(x, edge_index, W_gcn1, b_gcn1, g1, be1, W_gcn2, b_gcn2, g2, be2,
           Wih0, Whh0, bih0, bhh0, Wih1, Whh1, bih1, bhh1, Wc1, bc1, Wc2, bc2,
           Wr1, br1, Wr2, br2, We, bee, Wn, bn):
    x = x.astype(jnp.float32)
    src = edge_index[0].astype(jnp.int32)
    dst = edge_index[1].astype(jnp.int32)
    pidx = jnp.stack([src.reshape(NS, NCH, K), dst.reshape(NS, NCH, K)],
                     axis=2)
    zeros_1d = jnp.zeros((N,), jnp.float32)

    sc_agg_a = _make_sc_agg(0)
    sc_agg_b = _make_sc_agg(GN)
    sc_deg = _make_sc_deg()
    sc_root = _make_sc_root()

    cnt_p = sc_deg(dst, zeros_1d).reshape(NS, N, 1)
    cnt, inv = _tc_deg(cnt_p)

    xf = x.reshape(BT * N, D)
    agg1a = sc_agg_a(xf, pidx).reshape(GN, N, D)
    agg1b = sc_agg_b(xf, pidx).reshape(GN, N, D)
    h1a = _tc_dense(agg1a, inv, W_gcn1, b_gcn1, g1, be1)
    h1b = _tc_dense(agg1b, inv, W_gcn1, b_gcn1, g1, be1)
    agg2a = sc_agg_a(h1a.reshape(GN * N, H), pidx).reshape(GN, N, D)
    hga = _tc_gru(agg2a.reshape(1, T, N, D), inv, W_gcn2, b_gcn2, g2, be2,
                  Wih0, Whh0, bih0, bhh0, Wih1, Whh1, bih1, bhh1)
    agg2b = sc_agg_a(h1b.reshape(GN * N, H), pidx).reshape(GN, N, D)
    inv_b = inv + 0.0 * hga[0, :, :1]
    hgb = _tc_gru(agg2b.reshape(1, T, N, D), inv_b, W_gcn2, b_gcn2, g2, be2,
                  Wih0, Whh0, bih0, bhh0, Wih1, Whh1, bih1, bhh1)

    rc, rr, nsa, base = _tc_heads(hga, hgb, cnt, Wc1, bc1, Wc2, bc2,
                                  Wr1, br1, Wr2, br2, We, bee, Wn, bn)
    hg = jnp.concatenate([hga, hgb], axis=0)

    seg = sc_root(nsa.reshape(B, N), src, dst, zeros_1d)
    root = _tc_softmax(seg.reshape(B, NS, N), base.reshape(B, N))

    return rc, rr.reshape(B, N), root, hg


# speedup vs baseline: 1.0275x; 1.0063x over previous
import functools
import jax
import jax.numpy as jnp
from jax import lax
from jax.experimental import pallas as pl
from jax.experimental.pallas import tpu as pltpu
from jax.experimental.pallas import tpu_sc as plsc

B = 2
T = 8
BT = B * T
N = 10000
E = 160000
D = 128
H = 128

NC = 2
NS = 16
LANES = 16

EPT = E // NS
K = 80
NCH = EPT // K
GN = BT // 2
SPT = GN // NC
RPT = N // NS

_mesh = lambda: plsc.VectorSubcoreMesh(
    core_axis_name="c", subcore_axis_name="s", num_cores=NC, num_subcores=NS)


def _sc_agg_body(base, xf_hbm, pidx_hbm, out_hbm,
                 pidx_t, rb0, rb1, agg_sh,
                 g0, g1, s0, s1, zsem, csem):
    cid = lax.axis_index("c")
    sid = lax.axis_index("s")

    pltpu.sync_copy(pidx_hbm.at[sid], pidx_t)

    def slice_body(j, carry):
        lbt = cid * SPT + j
        bt = base + lbt
        r0 = sid * RPT
        xfsl = xf_hbm.at[pl.ds(bt * N, N)]

        def gth(k, rbuf, sem):
            return pltpu.make_async_copy(
                xfsl.at[pidx_t.at[k].at[0]], rbuf, sem)

        def sct(k, rbuf, sem):
            return pltpu.make_async_copy(
                rbuf, agg_sh.at[pidx_t.at[k].at[1]], sem)

        pltpu.make_async_copy(xfsl.at[pl.ds(r0, RPT)],
                              agg_sh.at[pl.ds(r0, RPT)], zsem).start()
        pltpu.make_async_copy(xfsl.at[pl.ds(r0, RPT)],
                              agg_sh.at[pl.ds(r0, RPT)], zsem).wait()
        plsc.subcore_barrier()

        gth(0, rb0, g0).start()
        gth(1, rb1, g1).start()

        def pair(i, c2):
            k0 = 2 * i
            k1 = 2 * i + 1
            gth(k0, rb0, g0).wait()
            sct(k0, rb0, s0).start(add=True)
            sct(k0, rb0, s0).wait()
            gth(k0 + 2, rb0, g0).start()
            gth(k1, rb1, g1).wait()
            sct(k1, rb1, s1).start(add=True)
            sct(k1, rb1, s1).wait()

            @pl.when(k1 + 2 < NCH)
            def _():
                gth(k1 + 2, rb1, g1).start()
            return c2
        lax.fori_loop(0, (NCH - 1) // 2, pair, 0)
        gth(NCH - 1, rb0, g0).wait()
        sct(NCH - 1, rb0, s0).start(add=True)
        sct(NCH - 1, rb0, s0).wait()
        plsc.subcore_barrier()

        pltpu.make_async_copy(agg_sh.at[pl.ds(r0, RPT)],
                              out_hbm.at[pl.ds(lbt * N + r0, RPT)],
                              csem).start()
        pltpu.make_async_copy(agg_sh.at[pl.ds(r0, RPT)],
                              out_hbm.at[pl.ds(lbt * N + r0, RPT)],
                              csem).wait()
        plsc.subcore_barrier()
        return carry

    lax.fori_loop(0, SPT, slice_body, 0)


def _make_sc_agg(base, interpret=False):
    return pl.kernel(
        functools.partial(_sc_agg_body, base),
        out_type=jax.ShapeDtypeStruct((GN * N, D), jnp.float32),
        mesh=_mesh(),
        compiler_params=pltpu.CompilerParams(needs_layout_passes=False, use_tc_tiling_on_sc=False),
        scratch_types=[
            pltpu.VMEM((NCH, 2, K), jnp.int32),
            pltpu.VMEM((K, D), jnp.float32),
            pltpu.VMEM((K, D), jnp.float32),
            pltpu.VMEM_SHARED((N, D), jnp.float32),
            pltpu.SemaphoreType.DMA,
            pltpu.SemaphoreType.DMA,
            pltpu.SemaphoreType.DMA,
            pltpu.SemaphoreType.DMA,
            pltpu.SemaphoreType.DMA,
            pltpu.SemaphoreType.DMA,
        ],
        interpret=interpret,
    )


def _sc_deg_body(dst_hbm, z1d_hbm, cnt_hbm, dst_all, deg1d):
    cid = lax.axis_index("c")
    sid = lax.axis_index("s")

    @pl.when(cid == 0)
    def _():
        pltpu.sync_copy(dst_hbm.at[pl.ds(sid * EPT, EPT)], dst_all)
        pltpu.sync_copy(z1d_hbm, deg1d)

        ones = jnp.full((LANES,), 1.0, jnp.float32)

        def it(i, c2):
            d = dst_all[pl.ds(i * LANES, LANES)]
            plsc.addupdate_scatter(deg1d, [d], ones)
            return c2
        lax.fori_loop(0, EPT // LANES, it, 0)
        pltpu.sync_copy(deg1d, cnt_hbm.at[sid])


def _make_sc_deg(interpret=False):
    return pl.kernel(
        _sc_deg_body,
        out_type=jax.ShapeDtypeStruct((NS, N), jnp.float32),
        mesh=_mesh(),
        compiler_params=pltpu.CompilerParams(needs_layout_passes=False, use_tc_tiling_on_sc=False),
        scratch_types=[
            pltpu.VMEM((EPT,), jnp.int32),
            pltpu.VMEM((N,), jnp.float32),
        ],
        interpret=interpret,
    )


def _sc_root_body(nsa_hbm, src_hbm, dst_hbm, z1d_hbm, seg_hbm,
                  nsa_v, src_all, dst_all, seg1d):
    cid = lax.axis_index("c")
    sid = lax.axis_index("s")

    pltpu.sync_copy(nsa_hbm.at[cid], nsa_v)
    pltpu.sync_copy(src_hbm.at[pl.ds(sid * EPT, EPT)], src_all)
    pltpu.sync_copy(dst_hbm.at[pl.ds(sid * EPT, EPT)], dst_all)
    pltpu.sync_copy(z1d_hbm, seg1d)

    def it(i, c2):
        s = src_all[pl.ds(i * LANES, LANES)]
        vals = plsc.load_gather(nsa_v, [s])
        d = dst_all[pl.ds(i * LANES, LANES)]
        plsc.addupdate_scatter(seg1d, [d], vals)
        return c2
    lax.fori_loop(0, EPT // LANES, it, 0)
    pltpu.sync_copy(seg1d, seg_hbm.at[cid * NS + sid])


def _make_sc_root(interpret=False):
    return pl.kernel(
        _sc_root_body,
        out_type=jax.ShapeDtypeStruct((B * NS, N), jnp.float32),
        mesh=_mesh(),
        compiler_params=pltpu.CompilerParams(needs_layout_passes=False, use_tc_tiling_on_sc=False),
        scratch_types=[
            pltpu.VMEM((N,), jnp.float32),
            pltpu.VMEM((EPT,), jnp.int32),
            pltpu.VMEM((EPT,), jnp.int32),
            pltpu.VMEM((N,), jnp.float32),
        ],
        interpret=interpret,
    )


_RB = 1000


def _deg_body(cnt_ref, c_ref, inv_ref):
    c = jnp.sum(cnt_ref[...], axis=0)
    c_ref[...] = c
    inv_ref[...] = 1.0 / (c + 1.0)


def _tc_deg(cnt_p, interpret=False):
    return pl.pallas_call(
        _deg_body,
        grid=(N // _RB,),
        in_specs=[pl.BlockSpec((NS, _RB, 1), lambda nb: (0, nb, 0))],
        out_specs=[
            pl.BlockSpec((_RB, 1), lambda nb: (nb, 0)),
            pl.BlockSpec((_RB, 1), lambda nb: (nb, 0)),
        ],
        out_shape=[
            jax.ShapeDtypeStruct((N, 1), jnp.float32),
            jax.ShapeDtypeStruct((N, 1), jnp.float32),
        ],
        interpret=interpret,
    )(cnt_p)


def _dense_body_dual(a_ref, inv_ref, wT_ref, b_ref, g_ref, be_ref,
                     o_ref, ob_ref):
    v = a_ref[0] * inv_ref[...]
    o = jnp.dot(v.astype(jnp.bfloat16), wT_ref[...],
                preferred_element_type=jnp.float32) + b_ref[...]
    mu = jnp.mean(o, axis=-1, keepdims=True)
    dlt = o - mu
    var = jnp.mean(dlt * dlt, axis=-1, keepdims=True)
    y = dlt * lax.rsqrt(var + 1e-5) * g_ref[...] + be_ref[...]
    r = jnp.maximum(y, 0.0)
    o_ref[0] = r
    if ob_ref is not None:
        ob_ref[0] = r.astype(jnp.bfloat16)


def _dense_body(a_ref, inv_ref, wT_ref, b_ref, g_ref, be_ref, o_ref):
    _dense_body_dual(a_ref, inv_ref, wT_ref, b_ref, g_ref, be_ref,
                     o_ref, None)


def _tc_dense(agg3, inv, W, b, g, be, dual=False, interpret=False):
    wspec = pl.BlockSpec((D, H), lambda bt, nb: (0, 0))
    vspec = pl.BlockSpec((1, H), lambda bt, nb: (0, 0))
    ospec = pl.BlockSpec((1, _RB, H), lambda bt, nb: (bt, nb, 0))
    nbt = agg3.shape[0]
    return pl.pallas_call(
        _dense_body_dual if dual else _dense_body,
        grid=(nbt, N // _RB),
        in_specs=[
            pl.BlockSpec((1, _RB, D), lambda bt, nb: (bt, nb, 0)),
            pl.BlockSpec((_RB, 1), lambda bt, nb: (nb, 0)),
            wspec, vspec, vspec, vspec,
        ],
        out_specs=[ospec, ospec] if dual else ospec,
        out_shape=(
            [jax.ShapeDtypeStruct((nbt, N, H), jnp.float32),
             jax.ShapeDtypeStruct((nbt, N, H), jnp.bfloat16)]
            if dual else jax.ShapeDtypeStruct((nbt, N, H), jnp.float32)),
        interpret=interpret,
    )(agg3, inv, W.T.astype(jnp.bfloat16), b[None, :], g[None, :], be[None, :])


def _gru_step(x, h, WiT, WhT, bi, bh):
    gi = jnp.dot(x.astype(jnp.bfloat16), WiT,
                 preferred_element_type=jnp.float32) + bi
    gh = jnp.dot(h.astype(jnp.bfloat16), WhT,
                 preferred_element_type=jnp.float32) + bh
    r = jax.nn.sigmoid(gi[:, :H] + gh[:, :H])
    z = jax.nn.sigmoid(gi[:, H:2 * H] + gh[:, H:2 * H])
    n = jnp.tanh(gi[:, 2 * H:] + r * gh[:, 2 * H:])
    return (1.0 - z) * n + z * h


def _gru_body(a_ref, inv_ref, cnt_ref, wT_ref, b_ref, g_ref, be_ref,
              wi0_ref, wh0_ref, bi0_ref, bh0_ref,
              wi1_ref, wh1_ref, bi1_ref, bh1_ref,
              wc1_ref, bc1_ref, wc2_ref, bc2_ref,
              wr1_ref, br1_ref, wr2_ref, br2_ref,
              we_ref, bee_ref, wn_ref, bn_ref,
              o_ref, rc_ref, rr_ref, nsa_ref, base_ref):
    inv = inv_ref[...]
    wT = wT_ref[...]
    bb = b_ref[...]
    gg = g_ref[...]
    be = be_ref[...]
    Wi0 = wi0_ref[...]
    Wh0 = wh0_ref[...]
    Wi1 = wi1_ref[...]
    Wh1 = wh1_ref[...]
    bi0 = bi0_ref[...]
    bh0 = bh0_ref[...]
    bi1 = bi1_ref[...]
    bh1 = bh1_ref[...]
    h0 = jnp.zeros((_RB, H), jnp.float32)
    h1 = jnp.zeros((_RB, H), jnp.float32)
    for t in range(T):
        v = a_ref[0, t] * inv
        o = jnp.dot(v.astype(jnp.bfloat16), wT,
                    preferred_element_type=jnp.float32) + bb
        mu = jnp.mean(o, axis=-1, keepdims=True)
        dlt = o - mu
        var = jnp.mean(dlt * dlt, axis=-1, keepdims=True)
        xt = jnp.maximum(dlt * lax.rsqrt(var + 1e-5) * gg + be, 0.0)
        h0 = _gru_step(xt, h0, Wi0, Wh0, bi0, bh0)
        h1 = _gru_step(h0, h1, Wi1, Wh1, bi1, bh1)
    o_ref[0] = h1
    c = cnt_ref[...]
    weT = we_ref[...]
    z1 = jnp.maximum(
        jnp.dot(h1, wc1_ref[...], preferred_element_type=jnp.float32)
        + bc1_ref[...], 0.0)
    rc_ref[0] = jnp.dot(z1, wc2_ref[...],
                        preferred_element_type=jnp.float32) + bc2_ref[...]
    z2 = jnp.maximum(
        jnp.dot(h1, wr1_ref[...], preferred_element_type=jnp.float32)
        + br1_ref[...], 0.0)
    rr_ref[0] = jax.nn.sigmoid(
        jnp.dot(z2, wr2_ref[...], preferred_element_type=jnp.float32)
        + br2_ref[...])
    nsa_ref[0] = jnp.dot(h1, weT[:H], preferred_element_type=jnp.float32)
    nsb = jnp.dot(h1, weT[H:], preferred_element_type=jnp.float32)
    nss = jnp.dot(h1, wn_ref[...],
                  preferred_element_type=jnp.float32) + bn_ref[...]
    base_ref[0] = c * (nsb + bee_ref[...]) + nss


def _tc_gru(a4, inv, cnt, W, b, g, be, Wih0, Whh0, bih0, bhh0,
            Wih1, Whh1, bih1, bhh1, Wc1, bc1, Wc2, bc2,
            Wr1, br1, Wr2, br2, We, bee, Wn, bn, interpret=False):
    wspec = pl.BlockSpec((H, 3 * H), lambda bq, nb: (0, 0))
    bspec = pl.BlockSpec((1, 3 * H), lambda bq, nb: (0, 0))

    def full(shape):
        return pl.BlockSpec(shape, lambda bq, nb: tuple(0 for _ in shape))

    def obs(w):
        return pl.BlockSpec((1, _RB, w), lambda bq, nb: (bq, nb, 0))
    return pl.pallas_call(
        _gru_body,
        grid=(1, N // _RB),
        in_specs=[
            pl.BlockSpec((1, T, _RB, D), lambda bq, nb: (bq, 0, nb, 0)),
            pl.BlockSpec((_RB, 1), lambda bq, nb: (nb, 0)),
            pl.BlockSpec((_RB, 1), lambda bq, nb: (nb, 0)),
            full((D, H)), full((1, H)), full((1, H)), full((1, H)),
            wspec, wspec, bspec, bspec, wspec, wspec, bspec, bspec,
            full((H, H // 2)), full((1, H // 2)),
            full((H // 2, 4)), full((1, 4)),
            full((H, H // 2)), full((1, H // 2)),
            full((H // 2, 1)), full((1, 1)),
            full((2 * H, 1)), full((1, 1)),
            full((H, 1)), full((1, 1)),
        ],
        out_specs=[obs(H), obs(4), obs(1), obs(1), obs(1)],
        out_shape=[
            jax.ShapeDtypeStruct((1, N, H), jnp.float32),
            jax.ShapeDtypeStruct((1, N, 4), jnp.float32),
            jax.ShapeDtypeStruct((1, N, 1), jnp.float32),
            jax.ShapeDtypeStruct((1, N, 1), jnp.float32),
            jax.ShapeDtypeStruct((1, N, 1), jnp.float32),
        ],
        interpret=interpret,
    )(a4, inv, cnt, W.T.astype(jnp.bfloat16), b[None, :], g[None, :],
      be[None, :],
      Wih0.T.astype(jnp.bfloat16), Whh0.T.astype(jnp.bfloat16),
      bih0[None, :], bhh0[None, :],
      Wih1.T.astype(jnp.bfloat16), Whh1.T.astype(jnp.bfloat16),
      bih1[None, :], bhh1[None, :],
      Wc1.T, bc1[None, :], Wc2.T, bc2[None, :],
      Wr1.T, br1[None, :], Wr2.T, br2[None, :],
      We.T, bee[None, :], Wn.T, bn[None, :])


def _softmax_body(seg_ref, base_ref, o_ref):
    l = jnp.sum(seg_ref[...], axis=1) + base_ref[...]
    m = jnp.max(l, axis=-1, keepdims=True)
    e = jnp.exp(l - m)
    o_ref[...] = e / jnp.sum(e, axis=-1, keepdims=True)


def _tc_softmax(seg, base, interpret=False):
    return pl.pallas_call(
        _softmax_body,
        out_shape=jax.ShapeDtypeStruct((B, N), jnp.float32),
        interpret=interpret,
    )(seg, base)


def kernel(x, edge_index, W_gcn1, b_gcn1, g1, be1, W_gcn2, b_gcn2, g2, be2,
           Wih0, Whh0, bih0, bhh0, Wih1, Whh1, bih1, bhh1, Wc1, bc1, Wc2, bc2,
           Wr1, br1, Wr2, br2, We, bee, Wn, bn):
    x = x.astype(jnp.float32)
    src = edge_index[0].astype(jnp.int32)
    dst = edge_index[1].astype(jnp.int32)
    pidx = jnp.stack([src.reshape(NS, NCH, K), dst.reshape(NS, NCH, K)],
                     axis=2)
    zeros_1d = jnp.zeros((N,), jnp.float32)

    sc_agg_a = _make_sc_agg(0)
    sc_agg_b = _make_sc_agg(GN)
    sc_deg = _make_sc_deg()
    sc_root = _make_sc_root()

    cnt_p = sc_deg(dst, zeros_1d).reshape(NS, N, 1)
    cnt, inv = _tc_deg(cnt_p)

    xf = x.reshape(BT * N, D)
    agg1a = sc_agg_a(xf, pidx).reshape(GN, N, D)
    agg1b = sc_agg_b(xf, pidx).reshape(GN, N, D)
    h1a = _tc_dense(agg1a, inv, W_gcn1, b_gcn1, g1, be1)
    h1b = _tc_dense(agg1b, inv, W_gcn1, b_gcn1, g1, be1)
    agg2a = sc_agg_a(h1a.reshape(GN * N, H), pidx).reshape(GN, N, D)
    hga, rca, rra, nsaa, basea = _tc_gru(
        agg2a.reshape(1, T, N, D), inv, cnt, W_gcn2, b_gcn2, g2, be2,
        Wih0, Whh0, bih0, bhh0, Wih1, Whh1, bih1, bhh1,
        Wc1, bc1, Wc2, bc2, Wr1, br1, Wr2, br2, We, bee, Wn, bn)
    agg2b = sc_agg_a(h1b.reshape(GN * N, H), pidx).reshape(GN, N, D)
    inv_b = inv + 0.0 * hga[0, :, :1]
    hgb, rcb, rrb, nsab, baseb = _tc_gru(
        agg2b.reshape(1, T, N, D), inv_b, cnt, W_gcn2, b_gcn2, g2, be2,
        Wih0, Whh0, bih0, bhh0, Wih1, Whh1, bih1, bhh1,
        Wc1, bc1, Wc2, bc2, Wr1, br1, Wr2, br2, We, bee, Wn, bn)

    rc = jnp.concatenate([rca, rcb], axis=0)
    rr = jnp.concatenate([rra, rrb], axis=0)
    nsa = jnp.concatenate([nsaa, nsab], axis=0)
    base = jnp.concatenate([basea, baseb], axis=0)
    hg = jnp.concatenate([hga, hgb], axis=0)

    seg = sc_root(nsa.reshape(B, N), src, dst, zeros_1d)
    root = _tc_softmax(seg.reshape(B, NS, N), base.reshape(B, N))

    return rc, rr.reshape(B, N), root, hg
